# baseline (device time: 13757 ns/iter reference)
import jax
import jax.numpy as jnp
from jax import lax
from jax.experimental import pallas as pl
from jax.experimental.pallas import tpu as pltpu


def kernel(partial, resid, gamma):
    _, M, D = partial.shape

    def body(partial_ref, resid_ref, gamma_ref, out_ref,
             send_buf, recv_buf, send_sem, recv_sem):
        my_x = lax.axis_index("x")
        my_y = lax.axis_index("y")
        nbr = (my_x, 1 - my_y)

        barrier_sem = pltpu.get_barrier_semaphore()
        pl.semaphore_signal(barrier_sem, inc=1, device_id=nbr,
                            device_id_type=pl.DeviceIdType.MESH)
        pl.semaphore_wait(barrier_sem, 1)

        send_buf[...] = partial_ref[0].astype(jnp.bfloat16)
        rdma = pltpu.make_async_remote_copy(
            src_ref=send_buf,
            dst_ref=recv_buf,
            send_sem=send_sem,
            recv_sem=recv_sem,
            device_id=nbr,
            device_id_type=pl.DeviceIdType.MESH,
        )
        rdma.start()
        rdma.wait()

        y = (partial_ref[0] + recv_buf[...].astype(jnp.float32)
             + resid_ref[...])
        rms = jnp.sqrt(jnp.mean(y * y, axis=-1, keepdims=True) + 1e-6)
        out_ref[...] = y / rms * gamma_ref[...]

    return pl.pallas_call(
        body,
        out_shape=jax.ShapeDtypeStruct((M, D), jnp.float32),
        in_specs=[pl.BlockSpec(memory_space=pltpu.VMEM)] * 3,
        out_specs=pl.BlockSpec(memory_space=pltpu.VMEM),
        scratch_shapes=[
            pltpu.VMEM((M, D), jnp.bfloat16),
            pltpu.VMEM((M, D), jnp.bfloat16),
            pltpu.SemaphoreType.DMA,
            pltpu.SemaphoreType.DMA,
        ],
        compiler_params=pltpu.CompilerParams(collective_id=0),
    )(partial, resid, gamma.reshape(1, D))
